# Initial kernel scaffold; baseline (speedup 1.0000x reference)
import jax
import jax.numpy as jnp
from jax.experimental import pallas as pl


def kernel(num_samples, loc, log_scale, weight_scores, rdn, eps, wo):
    raise NotImplementedError("write your pallas kernel here")


# speedup vs baseline: 7.2964x; 7.2964x over previous
import functools

import jax
import jax.numpy as jnp
import numpy as np
from jax.experimental import pallas as pl
from jax.experimental.pallas import tpu as pltpu

N_MODES = 16
PO2 = 2.0 * np.pi
PO4 = 4.0 * np.pi
INV_PI = 1.0 / np.pi


def _tc_body(consts_ref, rdn_ref, e0_ref, e1_ref, w0_ref, w1_ref,
             z0_ref, z1_ref, lp_ref):
    c = consts_ref
    wlast, ss0, ss1, ls0, ls1, lamb_in, lamb_out = (
        c[0], c[1], c[2], c[3], c[4], c[5], c[6])
    rdn = rdn_ref[...]
    w0 = w0_ref[...]
    w1 = w1_ref[...]
    mask = rdn < wlast
    cond1 = jnp.abs(w0) > jnp.abs(w1)
    zero_pos = jnp.logical_and(w0 == 0.0, w1 == 0.0)
    cond2 = jnp.logical_and(~cond1, ~zero_pos)
    d0 = jnp.where(w0 == 0.0, 1.0, w0)
    d1 = jnp.where(w1 == 0.0, 1.0, w1)
    num = jnp.where(cond1, w1, w0)
    den = jnp.where(cond1, d0, d1)
    t = PO4 * num / den
    theta = jnp.where(cond1, t, PO2 - t)
    r = jnp.where(cond1, w0, jnp.where(cond2, w1, 0.0))
    s0 = r * jnp.cos(theta)
    s1 = r * jnp.sin(theta)
    z0 = jnp.where(mask, s0, e0_ref[...] * ss0 + ls0)
    z1 = jnp.where(mask, s1, e1_ref[...] * ss1 + ls1)
    z0s = z0 * z0
    z1s = z1 * z1
    lps = [jnp.where(z0s + z1s >= 1.0, lamb_out, lamb_in)]
    for m in range(N_MODES):
        lps.append(c[7 + m] + c[23 + m] * z0 + c[39 + m] * z1
                   - c[55 + m] * z0s - c[71 + m] * z1s)
    mx = lps[0]
    for v in lps[1:]:
        mx = jnp.maximum(mx, v)
    s = jnp.exp(lps[0] - mx)
    for v in lps[1:]:
        s = s + jnp.exp(v - mx)
    z0_ref[...] = z0
    z1_ref[...] = z1
    lp_ref[...] = mx + jnp.log(s)


def _pack_consts(loc, log_scale, weight_scores):
    w = jax.nn.softmax(weight_scores, axis=1)[0]
    wlast = w[-1]
    lc = loc[0]
    sc = jnp.exp(log_scale[0])
    b = 0.5 / (sc * sc)
    a = (-0.5 * 2.0 * np.log(2.0 * np.pi)
         + jnp.log(w[:-1] + 1e-05) - log_scale[0].sum(axis=1))
    cc = a - b[:, 0] * lc[:, 0] ** 2 - b[:, 1] * lc[:, 1] ** 2
    p0 = 2.0 * b[:, 0] * lc[:, 0]
    p1 = 2.0 * b[:, 1] * lc[:, 1]
    ss = sc.sum(axis=0)
    ls = lc.sum(axis=0)
    lamb_in = jnp.log(INV_PI + 1e-05) + jnp.log(wlast)
    lamb_out = jnp.log(2e-05) + jnp.log(wlast)
    head = jnp.stack([wlast, ss[0], ss[1], ls[0], ls[1], lamb_in, lamb_out])
    return jnp.concatenate([head, cc, p0, p1, b[:, 0], b[:, 1]])


def kernel(num_samples, loc, log_scale, weight_scores, rdn, eps, wo):
    n = rdn.shape[0]
    consts = _pack_consts(loc, log_scale, weight_scores)
    e0, e1 = eps[:, 0], eps[:, 1]
    w0, w1 = wo[:, 0], wo[:, 1]
    blk = 65536
    grid = (n // blk,)
    vec = lambda: pl.BlockSpec((blk,), lambda i: (i,))
    z0, z1, lp = pl.pallas_call(
        _tc_body,
        grid=grid,
        in_specs=[pl.BlockSpec(memory_space=pltpu.SMEM)] + [vec() for _ in range(5)],
        out_specs=[vec() for _ in range(3)],
        out_shape=[jax.ShapeDtypeStruct((n,), jnp.float32) for _ in range(3)],
    )(consts, rdn, e0, e1, w0, w1)
    return jnp.stack([z0, z1], axis=1), lp
